# baseline (speedup 1.0000x reference)
import functools

import jax
import jax.numpy as jnp
from jax import lax
from jax.experimental import pallas as pl
from jax.experimental.pallas import tpu as pltpu
from jax.experimental.pallas import tpu_sc as plsc

GLOVE_DIM = 300
D_MODEL = 768
DIM_MAIN = 256
DIM_PAD = 384
TAIL_W = GLOVE_DIM - DIM_MAIN


def _tail_body(t_ref, o_ref):
    blk = t_ref[...]
    col = lax.broadcasted_iota(jnp.int32, blk.shape, 1)
    o_ref[...] = jnp.where(col < TAIL_W, blk, 0.0)


def _build_tail(table):
    v = table.shape[0]
    br = 1000
    return pl.pallas_call(
        _tail_body,
        grid=(v // br,),
        in_specs=[pl.BlockSpec((br, 128), lambda i: (i, 2))],
        out_specs=pl.BlockSpec((br, 128), lambda i: (i, 0)),
        out_shape=jax.ShapeDtypeStruct((v, 128), jnp.float32),
    )(table)


def _make_sc_gather(num_rows: int):
    info = plsc.get_sparse_core_info()
    nc, ns = info.num_cores, info.num_subcores
    nw = nc * ns
    assert num_rows % (8 * nw) == 0
    b_per_w = num_rows // nw
    chunk = 80
    assert b_per_w % chunk == 0 and chunk % 8 == 0
    n_chunks = b_per_w // chunk
    n_main = DIM_MAIN // 128

    mesh = plsc.VectorSubcoreMesh(core_axis_name="c", subcore_axis_name="s")

    @functools.partial(
        pl.kernel,
        mesh=mesh,
        out_type=jax.ShapeDtypeStruct((num_rows, DIM_PAD), jnp.float32),
        scratch_types=[
            pltpu.VMEM((2, chunk), jnp.int32),
            pltpu.VMEM((2, chunk, DIM_PAD), jnp.float32),
            pltpu.SemaphoreType.DMA,
            pltpu.SemaphoreType.DMA,
        ],
    )
    def gather(main_hbm, tail_hbm, idx_hbm, out_hbm, idx_v, rows_v, sem0, sem1):
        wid = lax.axis_index("s") * nc + lax.axis_index("c")
        base = wid * b_per_w
        sems = (sem0, sem1)

        def fire(g, buf):
            off = base + g * chunk
            pltpu.sync_copy(idx_hbm.at[pl.ds(off, chunk)], idx_v.at[buf])
            for t in range(n_main):
                pltpu.async_copy(
                    main_hbm.at[idx_v.at[buf], pl.ds(t * 128, 128)],
                    rows_v.at[buf, :, pl.ds(t * 128, 128)],
                    sems[buf],
                )
            pltpu.async_copy(
                tail_hbm.at[idx_v.at[buf]],
                rows_v.at[buf, :, pl.ds(DIM_MAIN, 128)],
                sems[buf],
            )

        def drain_write(g, buf):
            for _ in range(n_main + 1):
                pltpu.make_async_copy(
                    tail_hbm.at[idx_v.at[buf]],
                    rows_v.at[buf, :, pl.ds(DIM_MAIN, 128)],
                    sems[buf],
                ).wait()
            pltpu.sync_copy(rows_v.at[buf], out_hbm.at[pl.ds(base + g * chunk, chunk)])

        fire(0, 0)

        def body(t, _):
            g = 2 * t

            @pl.when(g + 1 < n_chunks)
            def _():
                fire(g + 1, 1)

            drain_write(g, 0)

            @pl.when(g + 1 < n_chunks)
            def _():
                @pl.when(g + 2 < n_chunks)
                def _():
                    fire(g + 2, 0)

                drain_write(g + 1, 1)

            return 0

        lax.fori_loop(0, (n_chunks + 1) // 2, body, 0)

    return gather


def _mm_body(a_ref, w_ref, b_ref, o_ref):
    o_ref[...] = (
        jnp.dot(
            a_ref[...].astype(jnp.bfloat16),
            w_ref[...].astype(jnp.bfloat16),
            preferred_element_type=jnp.float32,
        )
        + b_ref[...]
    )


def _matmul_tc(emb, wp, b):
    m = emb.shape[0]
    bm = 512
    return pl.pallas_call(
        _mm_body,
        grid=(m // bm,),
        in_specs=[
            pl.BlockSpec((bm, DIM_PAD), lambda i: (i, 0)),
            pl.BlockSpec((DIM_PAD, D_MODEL), lambda i: (0, 0)),
            pl.BlockSpec((1, D_MODEL), lambda i: (0, 0)),
        ],
        out_specs=pl.BlockSpec((bm, D_MODEL), lambda i: (i, 0)),
        out_shape=jax.ShapeDtypeStruct((m, D_MODEL), jnp.float32),
    )(emb, wp, b.reshape(1, D_MODEL))


def kernel(x, glove_table, W, b):
    batch, hist = x.shape
    idx = x.astype(jnp.int32).reshape(-1)
    tail = _build_tail(glove_table)
    wp = jnp.pad(W, ((0, DIM_PAD - GLOVE_DIM), (0, 0)))
    gather = _make_sc_gather(idx.shape[0])
    emb = gather(glove_table, tail, idx)
    out = _matmul_tc(emb, wp, b)
    return out.reshape(batch, hist, D_MODEL)


# speedup vs baseline: 1.3537x; 1.3537x over previous
import functools

import jax
import jax.numpy as jnp
from jax import lax
from jax.experimental import pallas as pl
from jax.experimental.pallas import tpu as pltpu
from jax.experimental.pallas import tpu_sc as plsc

GLOVE_DIM = 300
D_MODEL = 768
DIM_MAIN = 256
DIM_PAD = 384
TAIL_W = GLOVE_DIM - DIM_MAIN


def _build_tail(table):
    return jnp.pad(table[:, DIM_MAIN:], ((0, 0), (0, 128 - TAIL_W)))


def _make_sc_gather(num_rows: int):
    info = plsc.get_sparse_core_info()
    nc, ns = info.num_cores, info.num_subcores
    nw = nc * ns
    assert num_rows % (8 * nw) == 0
    b_per_w = num_rows // nw
    chunk = 80
    assert b_per_w % chunk == 0 and chunk % 8 == 0
    n_chunks = b_per_w // chunk
    n_main = DIM_MAIN // 128

    mesh = plsc.VectorSubcoreMesh(core_axis_name="c", subcore_axis_name="s")

    @functools.partial(
        pl.kernel,
        mesh=mesh,
        out_type=jax.ShapeDtypeStruct((num_rows, DIM_PAD), jnp.float32),
        scratch_types=[
            pltpu.VMEM((2, chunk), jnp.int32),
            pltpu.VMEM((2, chunk, DIM_PAD), jnp.float32),
            pltpu.SemaphoreType.DMA,
            pltpu.SemaphoreType.DMA,
        ],
    )
    def gather(main_hbm, tail_hbm, idx_hbm, out_hbm, idx_v, rows_v, sem0, sem1):
        wid = lax.axis_index("s") * nc + lax.axis_index("c")
        base = wid * b_per_w
        sems = (sem0, sem1)

        def fire(g, buf):
            off = base + g * chunk
            pltpu.sync_copy(idx_hbm.at[pl.ds(off, chunk)], idx_v.at[buf])
            for t in range(n_main):
                pltpu.async_copy(
                    main_hbm.at[idx_v.at[buf], pl.ds(t * 128, 128)],
                    rows_v.at[buf, :, pl.ds(t * 128, 128)],
                    sems[buf],
                )
            pltpu.async_copy(
                tail_hbm.at[idx_v.at[buf]],
                rows_v.at[buf, :, pl.ds(DIM_MAIN, 128)],
                sems[buf],
            )

        def drain_write(g, buf):
            for _ in range(n_main + 1):
                pltpu.make_async_copy(
                    tail_hbm.at[idx_v.at[buf]],
                    rows_v.at[buf, :, pl.ds(DIM_MAIN, 128)],
                    sems[buf],
                ).wait()
            pltpu.sync_copy(rows_v.at[buf], out_hbm.at[pl.ds(base + g * chunk, chunk)])

        fire(0, 0)

        def body(t, _):
            g = 2 * t

            @pl.when(g + 1 < n_chunks)
            def _():
                fire(g + 1, 1)

            drain_write(g, 0)

            @pl.when(g + 1 < n_chunks)
            def _():
                @pl.when(g + 2 < n_chunks)
                def _():
                    fire(g + 2, 0)

                drain_write(g + 1, 1)

            return 0

        lax.fori_loop(0, (n_chunks + 1) // 2, body, 0)

    return gather


def _mm_body(hist, a_ref, w_ref, b_ref, o_ref):
    res = (
        jnp.dot(
            a_ref[...].astype(jnp.bfloat16),
            w_ref[...].astype(jnp.bfloat16),
            preferred_element_type=jnp.float32,
        )
        + b_ref[...]
    )
    o_ref[...] = res.reshape(-1, hist, D_MODEL)


def _matmul_tc(emb, wp, b, batch, hist):
    bb = 16
    bm = bb * hist
    return pl.pallas_call(
        functools.partial(_mm_body, hist),
        grid=(batch // bb,),
        in_specs=[
            pl.BlockSpec((bm, DIM_PAD), lambda i: (i, 0)),
            pl.BlockSpec((DIM_PAD, D_MODEL), lambda i: (0, 0)),
            pl.BlockSpec((1, D_MODEL), lambda i: (0, 0)),
        ],
        out_specs=pl.BlockSpec((bb, hist, D_MODEL), lambda i: (i, 0, 0)),
        out_shape=jax.ShapeDtypeStruct((batch, hist, D_MODEL), jnp.float32),
    )(emb, wp, b.reshape(1, D_MODEL))


def kernel(x, glove_table, W, b):
    batch, hist = x.shape
    idx = x.astype(jnp.int32).reshape(-1)
    tail = _build_tail(glove_table)
    wp = jnp.pad(W, ((0, DIM_PAD - GLOVE_DIM), (0, 0)))
    gather = _make_sc_gather(idx.shape[0])
    emb = gather(glove_table, tail, idx)
    return _matmul_tc(emb, wp, b, batch, hist)


# speedup vs baseline: 1.8242x; 1.3476x over previous
import functools

import jax
import jax.numpy as jnp
from jax import lax
from jax.experimental import pallas as pl
from jax.experimental.pallas import tpu as pltpu
from jax.experimental.pallas import tpu_sc as plsc

GLOVE_DIM = 300
D_MODEL = 768
DIM_MAIN = 256
DIM_PAD = 384
TAIL_W = GLOVE_DIM - DIM_MAIN


def _build_tail(table):
    return jnp.pad(table[:, DIM_MAIN:], ((0, 0), (0, 128 - TAIL_W)))


def _make_sc_gather(num_rows: int):
    info = plsc.get_sparse_core_info()
    nc, ns = info.num_cores, info.num_subcores
    nw = nc * ns
    assert num_rows % (8 * nw) == 0
    b_per_w = num_rows // nw
    chunk = 80
    assert b_per_w % chunk == 0 and chunk % 8 == 0
    n_chunks = b_per_w // chunk
    n_main = DIM_MAIN // 128

    mesh = plsc.VectorSubcoreMesh(core_axis_name="c", subcore_axis_name="s")

    @functools.partial(
        pl.kernel,
        mesh=mesh,
        out_type=jax.ShapeDtypeStruct((num_rows, DIM_PAD), jnp.float32),
        scratch_types=[
            pltpu.VMEM((2, chunk), jnp.int32),
            pltpu.VMEM((2, chunk, DIM_PAD), jnp.float32),
            pltpu.SemaphoreType.DMA,
            pltpu.SemaphoreType.DMA,
        ],
    )
    def gather(main_hbm, tail_hbm, idx_hbm, out_hbm, idx_v, rows_v, sem0, sem1):
        wid = lax.axis_index("s") * nc + lax.axis_index("c")
        base = wid * b_per_w
        sems = (sem0, sem1)

        def fire(g, buf):
            off = base + g * chunk
            pltpu.sync_copy(idx_hbm.at[pl.ds(off, chunk)], idx_v.at[buf])
            for t in range(n_main):
                pltpu.async_copy(
                    main_hbm.at[idx_v.at[buf], pl.ds(t * 128, 128)],
                    rows_v.at[buf, :, pl.ds(t * 128, 128)],
                    sems[buf],
                )
            pltpu.async_copy(
                tail_hbm.at[idx_v.at[buf]],
                rows_v.at[buf, :, pl.ds(DIM_MAIN, 128)],
                sems[buf],
            )

        def drain_write(g, buf):
            for _ in range(n_main + 1):
                pltpu.make_async_copy(
                    tail_hbm.at[idx_v.at[buf]],
                    rows_v.at[buf, :, pl.ds(DIM_MAIN, 128)],
                    sems[buf],
                ).wait()
            pltpu.sync_copy(rows_v.at[buf], out_hbm.at[pl.ds(base + g * chunk, chunk)])

        fire(0, 0)

        def body(t, _):
            g = 2 * t

            @pl.when(g + 1 < n_chunks)
            def _():
                fire(g + 1, 1)

            drain_write(g, 0)

            @pl.when(g + 1 < n_chunks)
            def _():
                @pl.when(g + 2 < n_chunks)
                def _():
                    fire(g + 2, 0)

                drain_write(g + 1, 1)

            return 0

        lax.fori_loop(0, (n_chunks + 1) // 2, body, 0)

    return gather


def _mm_body(a_ref, w_ref, b_ref, o_ref):
    res = (
        jnp.dot(
            a_ref[...].astype(jnp.bfloat16),
            w_ref[...].astype(jnp.bfloat16),
            preferred_element_type=jnp.float32,
        )
        + b_ref[...]
    )
    o_ref[...] = res.reshape(1, -1, D_MODEL)


def _matmul_tc(emb, wp, b, batch, hist):
    return pl.pallas_call(
        _mm_body,
        grid=(hist,),
        in_specs=[
            pl.BlockSpec((batch, DIM_PAD), lambda i: (i, 0)),
            pl.BlockSpec((DIM_PAD, D_MODEL), lambda i: (0, 0)),
            pl.BlockSpec((1, D_MODEL), lambda i: (0, 0)),
        ],
        out_specs=pl.BlockSpec((1, batch, D_MODEL), lambda i: (i, 0, 0)),
        out_shape=jax.ShapeDtypeStruct((hist, batch, D_MODEL), jnp.float32),
    )(emb, wp, b.reshape(1, D_MODEL))


def kernel(x, glove_table, W, b):
    batch, hist = x.shape
    idx = x.T.astype(jnp.int32).reshape(-1)
    tail = _build_tail(glove_table)
    wp = jnp.pad(W, ((0, DIM_PAD - GLOVE_DIM), (0, 0)))
    gather = _make_sc_gather(idx.shape[0])
    emb = gather(glove_table, tail, idx)
    out_t = _matmul_tc(emb, wp, b, batch, hist)
    return jnp.transpose(out_t, (1, 0, 2))
